# speedup vs baseline: 1.2884x; 1.2884x over previous
import jax
import jax.numpy as jnp
from jax.experimental import pallas as pl

_CB = 256
_NCB = 16
_TOKENS = 512


def _encode_kernel(x_ref, wt_ref, b_ref, out_ref):
    x = x_ref[...]
    wt = wt_ref[...]
    logits = jax.lax.dot_general(
        x, wt, (((1,), (0,)), ((), ())), preferred_element_type=jnp.float32
    )
    logits = logits + b_ref[...]
    t = logits.shape[0]
    iota = jax.lax.broadcasted_iota(jnp.int32, (t, _CB), 1)
    cols = []
    for j in range(_NCB):
        s = logits[:, j * _CB : (j + 1) * _CB]
        m = jnp.max(s, axis=1, keepdims=True)
        idx = jnp.min(jnp.where(s == m, iota, _CB), axis=1, keepdims=True)
        cols.append(idx)
    out_ref[...] = jnp.concatenate(cols, axis=1)


def kernel(x, W, b):
    batch, hw, dim = x.shape
    n = W.shape[0]
    tokens = batch * hw
    xf = x.reshape(tokens, dim)
    wt = W.T
    b2 = b.reshape(1, n)
    out = pl.pallas_call(
        _encode_kernel,
        grid=(tokens // _TOKENS,),
        in_specs=[
            pl.BlockSpec((_TOKENS, dim), lambda i: (i, 0)),
            pl.BlockSpec((dim, n), lambda i: (0, 0)),
            pl.BlockSpec((1, n), lambda i: (0, 0)),
        ],
        out_specs=pl.BlockSpec((_TOKENS, _NCB), lambda i: (i, 0)),
        out_shape=jax.ShapeDtypeStruct((tokens, _NCB), jnp.int32),
    )(xf, wt, b2)
    return out.astype(jnp.uint8).reshape(batch, hw, _NCB)


# speedup vs baseline: 2.0670x; 1.6043x over previous
import jax
import jax.numpy as jnp
from jax.experimental import pallas as pl

_CB = 256
_NCB = 16
_TOKENS = 512


def _encode_kernel(x_ref, wt_ref, out_ref):
    x = x_ref[...]
    wt = wt_ref[...]
    logits = jax.lax.dot_general(
        x, wt, (((1,), (0,)), ((), ())), preferred_element_type=jnp.float32
    )
    t = logits.shape[0]
    iota = jax.lax.broadcasted_iota(jnp.int32, (t, _CB), 1).astype(jnp.float32)
    cols = []
    for j in range(_NCB):
        s = logits[:, j * _CB : (j + 1) * _CB]
        m = jnp.max(s, axis=1, keepdims=True)
        idx = jnp.min(jnp.where(s == m, iota, float(_CB)), axis=1, keepdims=True)
        cols.append(idx)
    out_ref[...] = jnp.concatenate(cols, axis=1).astype(jnp.int32)


def kernel(x, W, b):
    batch, hw, dim = x.shape
    n = W.shape[0]
    tokens = batch * hw
    xf = x.reshape(tokens, dim)
    xa = jnp.concatenate([xf, jnp.ones((tokens, 1), jnp.float32)], axis=1)
    wa = jnp.concatenate([W.T, b.reshape(1, n)], axis=0)
    out = pl.pallas_call(
        _encode_kernel,
        grid=(tokens // _TOKENS,),
        in_specs=[
            pl.BlockSpec((_TOKENS, dim + 1), lambda i: (i, 0)),
            pl.BlockSpec((dim + 1, n), lambda i: (0, 0)),
        ],
        out_specs=pl.BlockSpec((_TOKENS, _NCB), lambda i: (i, 0)),
        out_shape=jax.ShapeDtypeStruct((tokens, _NCB), jnp.int32),
    )(xa, wa)
    return out.astype(jnp.uint8).reshape(batch, hw, _NCB)


# speedup vs baseline: 2.2374x; 1.0825x over previous
import jax
import jax.numpy as jnp
from jax.experimental import pallas as pl
from jax.experimental.pallas import tpu as pltpu

_CB = 256
_NCB = 16
_TOKENS = 1024


def _encode_kernel(x_ref, wt_ref, out_ref):
    x = x_ref[...]
    wt = wt_ref[...]
    logits = jax.lax.dot_general(
        x, wt, (((1,), (0,)), ((), ())), preferred_element_type=jnp.float32
    )
    t = logits.shape[0]
    iota = jax.lax.broadcasted_iota(jnp.int32, (t, _CB), 1).astype(jnp.float32)
    cols = []
    for j in range(_NCB):
        s = logits[:, j * _CB : (j + 1) * _CB]
        m = jnp.max(s, axis=1, keepdims=True)
        idx = jnp.min(jnp.where(s == m, iota, float(_CB)), axis=1, keepdims=True)
        cols.append(idx)
    out_ref[...] = jnp.concatenate(cols, axis=1).astype(jnp.int32)


def kernel(x, W, b):
    batch, hw, dim = x.shape
    n = W.shape[0]
    tokens = batch * hw
    xf = x.reshape(tokens, dim)
    xa = jnp.concatenate([xf, jnp.ones((tokens, 1), jnp.float32)], axis=1)
    wa = jnp.concatenate([W.T, b.reshape(1, n)], axis=0)
    out = pl.pallas_call(
        _encode_kernel,
        grid=(tokens // _TOKENS,),
        in_specs=[
            pl.BlockSpec((_TOKENS, dim + 1), lambda i: (i, 0)),
            pl.BlockSpec((dim + 1, n), lambda i: (0, 0)),
        ],
        out_specs=pl.BlockSpec((_TOKENS, _NCB), lambda i: (i, 0)),
        compiler_params=pltpu.CompilerParams(dimension_semantics=("parallel",)),
        out_shape=jax.ShapeDtypeStruct((tokens, _NCB), jnp.int32),
    )(xa, wa)
    return out.astype(jnp.uint8).reshape(batch, hw, _NCB)
